# baseline (speedup 1.0000x reference)
import functools

import jax
import jax.numpy as jnp
from jax import lax
from jax.experimental import pallas as pl
from jax.experimental.pallas import tpu as pltpu
from jax.experimental.pallas import tpu_sc as plsc

D = 64
L = 16
NC, NS = 2, 16
NW = NC * NS
G = 128
GPC = 5
K = G * GPC
CC = 384


def _mesh():
    return plsc.VectorSubcoreMesh(
        core_axis_name="c", subcore_axis_name="s", num_cores=NC, num_subcores=NS
    )


@functools.lru_cache(maxsize=None)
def _build_conv(nrows):
    n_main = (nrows // G) * G
    n_tail = nrows - n_main
    nch = n_main // CC
    assert n_main % CC == 0

    W = D // 2

    @functools.partial(
        pl.kernel,
        out_type=jax.ShapeDtypeStruct((nrows * W,), jnp.int32),
        mesh=_mesh(),
        scratch_types=[
            pltpu.VMEM((D, CC), jnp.float32),
            pltpu.VMEM((D, CC), jnp.float32),
            pltpu.VMEM((CC * W,), jnp.int32),
            pltpu.VMEM((CC * W,), jnp.int32),
            pltpu.VMEM((D, G), jnp.float32),
            pltpu.VMEM((G * W,), jnp.int32),
            pltpu.SemaphoreType.DMA,
            pltpu.SemaphoreType.DMA,
        ],
        compiler_params=pltpu.CompilerParams(
            use_tc_tiling_on_sc=True, needs_layout_passes=False
        ),
    )
    def conv_kernel(src_hbm, tail_hbm, dst_hbm,
                    vin_a, vin_b, vout_a, vout_b, tin, tout, sem_i, sem_o):
        wid = lax.axis_index("s") * NC + lax.axis_index("c")
        base_n = nch // NW
        extra = nch - base_n * NW
        nw = base_n + jnp.where(wid < extra, 1, 0)
        iota = lax.iota(jnp.int32, L)
        rows = [iota + k * L for k in range(D // L)]
        perms = [jnp.bitwise_and(iota + s, L - 1) for s in range(L)]
        pvs = [perms[s] * W + iota for s in range(L)]

        def transpose_slab(vin, vout):
            @pl.loop(0, CC // L, unroll=4)
            def _cb(cb):
                c0 = cb * L
                c0w = c0 * W
                for s in range(L):
                    cols = perms[s] + c0
                    for k in range(W // L):
                        va = plsc.load_gather(vin, [rows[k], cols])
                        vb = plsc.load_gather(vin, [rows[k + 2], cols])
                        pk = plsc.pack(va, vb, format=plsc.PackFormat.INTERLEAVED)
                        w = plsc.bitcast(pk, jnp.int32)
                        sidx = pvs[s] + (c0w + k * L)
                        plsc.store_scatter(vout, [sidx], w)

        def issue_in(ch, vin):
            pltpu.async_copy(src_hbm.at[:, pl.ds(ch * CC, CC)], vin, sem_i)

        def wait_in(vin):
            pltpu.make_async_copy(src_hbm.at[:, pl.ds(0, CC)], vin, sem_i).wait()

        def issue_out(ch, vout):
            pltpu.async_copy(vout, dst_hbm.at[pl.ds(ch * CC * W, CC * W)], sem_o)

        def drain_out(vout):
            pltpu.make_async_copy(
                dst_hbm.at[pl.ds(0, CC * W)], vout, sem_o
            ).wait()

        @pl.when(nw > 0)
        def _prime():
            issue_in(wid, vin_a)

        @pl.when(nw > 1)
        def _prime2():
            issue_in(wid + NW, vin_b)

        @pl.loop(0, (nch // NW + 1 + 1) // 2)
        def _pair(p):
            ka = 2 * p
            kb = 2 * p + 1

            @pl.when(ka < nw)
            def _even():
                wait_in(vin_a)
                @pl.when(ka >= 2)
                def _():
                    drain_out(vout_a)
                transpose_slab(vin_a, vout_a)
                issue_out(wid + ka * NW, vout_a)
                @pl.when(ka + 2 < nw)
                def _():
                    issue_in(wid + (ka + 2) * NW, vin_a)

            @pl.when(kb < nw)
            def _odd():
                wait_in(vin_b)
                @pl.when(kb >= 2)
                def _():
                    drain_out(vout_b)
                transpose_slab(vin_b, vout_b)
                issue_out(wid + kb * NW, vout_b)
                @pl.when(kb + 2 < nw)
                def _():
                    issue_in(wid + (kb + 2) * NW, vin_b)

        @pl.when(nw > 0)
        def _drain1():
            drain_out(vout_a)

        @pl.when(nw > 1)
        def _drain2():
            drain_out(vout_b)

        if n_tail:
            @pl.when(wid == 0)
            def _tail():
                pltpu.sync_copy(tail_hbm, tin)

                @pl.loop(0, n_tail // L)
                def _cb(cb):
                    c0 = cb * L
                    for s in range(L):
                        cols = perms[s] + c0
                        for k in range(W // L):
                            va = plsc.load_gather(tin, [rows[k], cols])
                            vb = plsc.load_gather(tin, [rows[k + 2], cols])
                            pk = plsc.pack(
                                va, vb, format=plsc.PackFormat.INTERLEAVED
                            )
                            w = plsc.bitcast(pk, jnp.int32)
                            sidx = pvs[s] + (c0 * W + k * L)
                            plsc.store_scatter(tout, [sidx], w)

                pltpu.sync_copy(
                    tout.at[pl.ds(0, n_tail * W)],
                    dst_hbm.at[pl.ds(n_main * W, n_tail * W)],
                )

    return conv_kernel


@functools.lru_cache(maxsize=None)
def _build_gather(ntok):
    n_per_w = ntok // NW
    nchunks = n_per_w // K
    assert ntok % NW == 0 and n_per_w % K == 0

    assert nchunks % 2 == 0

    @functools.partial(
        pl.kernel,
        out_type=jax.ShapeDtypeStruct((ntok, D), jnp.float32),
        mesh=_mesh(),
        scratch_types=[
            pltpu.VMEM((GPC, G), jnp.int32),
            pltpu.VMEM((GPC, G), jnp.int32),
            pltpu.VMEM((GPC, G), jnp.int32),
            pltpu.VMEM((GPC, G), jnp.int32),
            pltpu.VMEM((K, D // 2), jnp.int32),
            pltpu.VMEM((K, D // 2), jnp.int32),
            pltpu.VMEM((K, D), jnp.float32),
            pltpu.SemaphoreType.DMA,
            pltpu.SemaphoreType.DMA,
            pltpu.SemaphoreType.DMA,
        ],
        compiler_params=pltpu.CompilerParams(
            use_tc_tiling_on_sc=False, needs_layout_passes=False
        ),
    )
    def sc_kernel(ids_hbm, ca_hbm, cent_hbm, off_hbm, out_hbm,
                  ids_a, ids_b, cids_a, cids_b, acc_a, acc_b, cen_v,
                  sem_a, sem_b, sem_c):
        wid = lax.axis_index("s") * NC + lax.axis_index("c")
        wbase = wid * n_per_w

        def stage(ids_v, cids_v, acc_v, c):
            base = wbase + c * K
            for j in range(GPC):
                pltpu.sync_copy(ids_hbm.at[pl.ds(base + j * G, G)], ids_v.at[j])
            for j in range(GPC):
                pltpu.async_copy(ca_hbm.at[ids_v.at[j]], cids_v.at[j], sem_a)
            for j in range(GPC):
                pltpu.async_copy(
                    off_hbm.at[ids_v.at[j]], acc_v.at[pl.ds(j * G, G)], sem_b
                )

        def drain(sem, kind):
            for j in range(GPC):
                if kind == "off":
                    src, dst = off_hbm.at[pl.ds(0, G)], acc_a.at[pl.ds(0, G)]
                elif kind == "cen":
                    src, dst = cent_hbm.at[pl.ds(0, G)], cen_v.at[pl.ds(0, G)]
                else:
                    src, dst = ca_hbm.at[pl.ds(0, G)], cids_a.at[0]
                pltpu.make_async_copy(src, dst, sem).wait()

        def finish(ids_v, cids_v, acc_v, c, nxt, guard):
            drain(sem_a, "ca")
            for j in range(GPC):
                pltpu.async_copy(
                    cent_hbm.at[cids_v.at[j]], cen_v.at[pl.ds(j * G, G)], sem_c
                )

            if guard is True:
                stage(*nxt)
            else:
                @pl.when(guard)
                def _stage_next():
                    stage(*nxt)

            drain(sem_b, "off")
            drain(sem_c, "cen")

            @pl.loop(0, K, unroll=8)
            def _add(t):
                for m in range(2):
                    vw = acc_v[t, pl.ds(m * L, L)]
                    bfv = plsc.bitcast(vw, jnp.bfloat16)
                    oa, ob = plsc.unpack(
                        bfv, format=plsc.PackFormat.INTERLEAVED
                    )
                    sa = pl.ds(m * L, L)
                    sb = pl.ds((m + 2) * L, L)
                    cen_v[t, sa] = cen_v[t, sa] + oa
                    cen_v[t, sb] = cen_v[t, sb] + ob

            pltpu.sync_copy(cen_v, out_hbm.at[pl.ds(wbase + c * K, K)])

        stage(ids_a, cids_a, acc_a, 0)
        npairs = nchunks // 2

        @pl.loop(0, npairs)
        def _pair(p):
            ca = 2 * p
            cb = 2 * p + 1
            finish(ids_a, cids_a, acc_a, ca,
                   (ids_b, cids_b, acc_b, cb), True)
            finish(ids_b, cids_b, acc_b, cb,
                   (ids_a, cids_a, acc_a, cb + 1), p < npairs - 1)

    return sc_kernel


def kernel(input_ids, cluster_assign, centroids, offsets):
    b, t = input_ids.shape
    v = offsets.shape[0]
    ids = input_ids.reshape(-1)
    off_t = offsets.T
    n_main = (v // G) * G
    tail = jnp.pad(offsets[n_main:].T, ((0, 0), (0, G - (v - n_main))))
    flat = _build_conv(v)(off_t, tail)
    off_lin = flat.reshape(v, D // 2)
    out = _build_gather(ids.shape[0])(ids, cluster_assign, centroids, off_lin)
    return out.reshape(b, t, D)


# speedup vs baseline: 1.0307x; 1.0307x over previous
import functools

import jax
import jax.numpy as jnp
from jax import lax
from jax.experimental import pallas as pl
from jax.experimental.pallas import tpu as pltpu
from jax.experimental.pallas import tpu_sc as plsc

D = 64
L = 16
NC, NS = 2, 16
NW = NC * NS
G = 128
GPC = 5
K = G * GPC
CC = 384


def _mesh():
    return plsc.VectorSubcoreMesh(
        core_axis_name="c", subcore_axis_name="s", num_cores=NC, num_subcores=NS
    )


@functools.lru_cache(maxsize=None)
def _build_conv(nrows):
    n_main = (nrows // G) * G
    n_tail = nrows - n_main
    nch = n_main // CC
    assert n_main % CC == 0

    W = D // 2

    @functools.partial(
        pl.kernel,
        out_type=jax.ShapeDtypeStruct((nrows * W,), jnp.int32),
        mesh=_mesh(),
        scratch_types=[
            pltpu.VMEM((D, CC), jnp.float32),
            pltpu.VMEM((D, CC), jnp.float32),
            pltpu.VMEM((CC * W,), jnp.int32),
            pltpu.VMEM((CC * W,), jnp.int32),
            pltpu.VMEM((D, G), jnp.float32),
            pltpu.VMEM((G * W,), jnp.int32),
            pltpu.SemaphoreType.DMA,
            pltpu.SemaphoreType.DMA,
        ],
        compiler_params=pltpu.CompilerParams(
            use_tc_tiling_on_sc=True, needs_layout_passes=False
        ),
    )
    def conv_kernel(src_hbm, tail_hbm, dst_hbm,
                    vin_a, vin_b, vout_a, vout_b, tin, tout, sem_i, sem_o):
        wid = lax.axis_index("s") * NC + lax.axis_index("c")
        base_n = nch // NW
        extra = nch - base_n * NW
        nw = base_n + jnp.where(wid < extra, 1, 0)
        iota = lax.iota(jnp.int32, L)
        rows = [iota + k * L for k in range(D // L)]
        perms = [jnp.bitwise_and(iota + s, L - 1) for s in range(L)]
        pvs = [perms[s] * W + iota for s in range(L)]

        def transpose_slab(vin, vout):
            @pl.loop(0, CC // L, unroll=2)
            def _cb(cb):
                c0 = cb * L
                c0w = c0 * W
                for s in range(L):
                    cols = perms[s] + c0
                    for k in range(W // L):
                        va = plsc.load_gather(vin, [rows[k], cols])
                        vb = plsc.load_gather(vin, [rows[k + 2], cols])
                        pk = plsc.pack(va, vb, format=plsc.PackFormat.INTERLEAVED)
                        w = plsc.bitcast(pk, jnp.int32)
                        sidx = pvs[s] + (c0w + k * L)
                        plsc.store_scatter(vout, [sidx], w)

        def issue_in(ch, vin):
            pltpu.async_copy(src_hbm.at[:, pl.ds(ch * CC, CC)], vin, sem_i)

        def wait_in(vin):
            pltpu.make_async_copy(src_hbm.at[:, pl.ds(0, CC)], vin, sem_i).wait()

        def issue_out(ch, vout):
            pltpu.async_copy(vout, dst_hbm.at[pl.ds(ch * CC * W, CC * W)], sem_o)

        def drain_out(vout):
            pltpu.make_async_copy(
                dst_hbm.at[pl.ds(0, CC * W)], vout, sem_o
            ).wait()

        @pl.when(nw > 0)
        def _prime():
            issue_in(wid, vin_a)

        @pl.when(nw > 1)
        def _prime2():
            issue_in(wid + NW, vin_b)

        @pl.loop(0, (nch // NW + 1 + 1) // 2)
        def _pair(p):
            ka = 2 * p
            kb = 2 * p + 1

            @pl.when(ka < nw)
            def _even():
                wait_in(vin_a)
                @pl.when(ka >= 2)
                def _():
                    drain_out(vout_a)
                transpose_slab(vin_a, vout_a)
                issue_out(wid + ka * NW, vout_a)
                @pl.when(ka + 2 < nw)
                def _():
                    issue_in(wid + (ka + 2) * NW, vin_a)

            @pl.when(kb < nw)
            def _odd():
                wait_in(vin_b)
                @pl.when(kb >= 2)
                def _():
                    drain_out(vout_b)
                transpose_slab(vin_b, vout_b)
                issue_out(wid + kb * NW, vout_b)
                @pl.when(kb + 2 < nw)
                def _():
                    issue_in(wid + (kb + 2) * NW, vin_b)

        @pl.when(nw > 0)
        def _drain1():
            drain_out(vout_a)

        @pl.when(nw > 1)
        def _drain2():
            drain_out(vout_b)

        if n_tail:
            @pl.when(wid == 0)
            def _tail():
                pltpu.sync_copy(tail_hbm, tin)

                @pl.loop(0, n_tail // L)
                def _cb(cb):
                    c0 = cb * L
                    for s in range(L):
                        cols = perms[s] + c0
                        for k in range(W // L):
                            va = plsc.load_gather(tin, [rows[k], cols])
                            vb = plsc.load_gather(tin, [rows[k + 2], cols])
                            pk = plsc.pack(
                                va, vb, format=plsc.PackFormat.INTERLEAVED
                            )
                            w = plsc.bitcast(pk, jnp.int32)
                            sidx = pvs[s] + (c0 * W + k * L)
                            plsc.store_scatter(tout, [sidx], w)

                pltpu.sync_copy(
                    tout.at[pl.ds(0, n_tail * W)],
                    dst_hbm.at[pl.ds(n_main * W, n_tail * W)],
                )

    return conv_kernel


@functools.lru_cache(maxsize=None)
def _build_gather(ntok):
    n_per_w = ntok // NW
    nchunks = n_per_w // K
    assert ntok % NW == 0 and n_per_w % K == 0

    assert nchunks % 2 == 0

    @functools.partial(
        pl.kernel,
        out_type=jax.ShapeDtypeStruct((ntok, D), jnp.float32),
        mesh=_mesh(),
        scratch_types=[
            pltpu.VMEM((GPC, G), jnp.int32),
            pltpu.VMEM((GPC, G), jnp.int32),
            pltpu.VMEM((GPC, G), jnp.int32),
            pltpu.VMEM((GPC, G), jnp.int32),
            pltpu.VMEM((K, D // 2), jnp.int32),
            pltpu.VMEM((K, D // 2), jnp.int32),
            pltpu.VMEM((K, D), jnp.float32),
            pltpu.SemaphoreType.DMA,
            pltpu.SemaphoreType.DMA,
            pltpu.SemaphoreType.DMA,
        ],
        compiler_params=pltpu.CompilerParams(
            use_tc_tiling_on_sc=False, needs_layout_passes=False
        ),
    )
    def sc_kernel(ids_hbm, ca_hbm, cent_hbm, off_hbm, out_hbm,
                  ids_a, ids_b, cids_a, cids_b, acc_a, acc_b, cen_v,
                  sem_a, sem_b, sem_c):
        wid = lax.axis_index("s") * NC + lax.axis_index("c")
        wbase = wid * n_per_w

        def stage(ids_v, cids_v, acc_v, c):
            base = wbase + c * K
            for j in range(GPC):
                pltpu.sync_copy(ids_hbm.at[pl.ds(base + j * G, G)], ids_v.at[j])
            for j in range(GPC):
                pltpu.async_copy(ca_hbm.at[ids_v.at[j]], cids_v.at[j], sem_a)
            for j in range(GPC):
                pltpu.async_copy(
                    off_hbm.at[ids_v.at[j]], acc_v.at[pl.ds(j * G, G)], sem_b
                )

        def drain(sem, kind):
            for j in range(GPC):
                if kind == "off":
                    src, dst = off_hbm.at[pl.ds(0, G)], acc_a.at[pl.ds(0, G)]
                elif kind == "cen":
                    src, dst = cent_hbm.at[pl.ds(0, G)], cen_v.at[pl.ds(0, G)]
                else:
                    src, dst = ca_hbm.at[pl.ds(0, G)], cids_a.at[0]
                pltpu.make_async_copy(src, dst, sem).wait()

        def finish(ids_v, cids_v, acc_v, c, nxt, guard):
            drain(sem_a, "ca")
            for j in range(GPC):
                pltpu.async_copy(
                    cent_hbm.at[cids_v.at[j]], cen_v.at[pl.ds(j * G, G)], sem_c
                )

            if guard is True:
                stage(*nxt)
            else:
                @pl.when(guard)
                def _stage_next():
                    stage(*nxt)

            drain(sem_b, "off")
            drain(sem_c, "cen")

            @pl.loop(0, K, unroll=4)
            def _add(t):
                for m in range(2):
                    vw = acc_v[t, pl.ds(m * L, L)]
                    oa = plsc.bitcast(lax.shift_left(vw, 16), jnp.float32)
                    ob = plsc.bitcast(
                        jnp.bitwise_and(vw, jnp.int32(-65536)), jnp.float32
                    )
                    sa = pl.ds(m * L, L)
                    sb = pl.ds((m + 2) * L, L)
                    cen_v[t, sa] = cen_v[t, sa] + oa
                    cen_v[t, sb] = cen_v[t, sb] + ob

            pltpu.sync_copy(cen_v, out_hbm.at[pl.ds(wbase + c * K, K)])

        stage(ids_a, cids_a, acc_a, 0)
        npairs = nchunks // 2

        @pl.loop(0, npairs)
        def _pair(p):
            ca = 2 * p
            cb = 2 * p + 1
            finish(ids_a, cids_a, acc_a, ca,
                   (ids_b, cids_b, acc_b, cb), True)
            finish(ids_b, cids_b, acc_b, cb,
                   (ids_a, cids_a, acc_a, cb + 1), p < npairs - 1)

    return sc_kernel


def kernel(input_ids, cluster_assign, centroids, offsets):
    b, t = input_ids.shape
    v = offsets.shape[0]
    ids = input_ids.reshape(-1)
    off_t = offsets.T
    n_main = (v // G) * G
    tail = jnp.pad(offsets[n_main:].T, ((0, 0), (0, G - (v - n_main))))
    flat = _build_conv(v)(off_t, tail)
    off_lin = flat.reshape(v, D // 2)
    out = _build_gather(ids.shape[0])(ids, cluster_assign, centroids, off_lin)
    return out.reshape(b, t, D)
